# baseline (speedup 1.0000x reference)
import functools

import jax
import jax.numpy as jnp
from jax import lax
from jax.experimental import pallas as pl
from jax.experimental.pallas import tpu as pltpu

_NUM_EXPERTS = 64
_TOP_K = 8
_BLOCK_T = 512


def _router_body(x_ref, w_ref, vals_ref, idx_ref):
    x = x_ref[...]
    w = w_ref[...]
    logits = lax.dot_general(
        x, w, (((1,), (1,)), ((), ())), preferred_element_type=jnp.float32
    )
    col = lax.broadcasted_iota(jnp.int32, logits.shape, 1)
    l = logits
    top_v = []
    top_i = []
    for _ in range(_TOP_K):
        m = jnp.max(l, axis=1, keepdims=True)
        hit = l >= m
        idx = jnp.min(jnp.where(hit, col, _NUM_EXPERTS), axis=1, keepdims=True)
        top_v.append(m)
        top_i.append(idx)
        l = jnp.where(col == idx, -jnp.inf, l)
    exps = [jnp.exp(v - top_v[0]) for v in top_v]
    denom = exps[0]
    for e in exps[1:]:
        denom = denom + e
    inv = 1.0 / denom
    for k in range(_TOP_K):
        vals_ref[:, k : k + 1] = exps[k] * inv
        idx_ref[:, k : k + 1] = top_i[k]


@jax.jit
def kernel(hidden_states, weight):
    tokens, hidden = hidden_states.shape
    grid = (tokens // _BLOCK_T,)
    vals, idx = pl.pallas_call(
        _router_body,
        grid=grid,
        in_specs=[
            pl.BlockSpec((_BLOCK_T, hidden), lambda i: (i, 0)),
            pl.BlockSpec((_NUM_EXPERTS, hidden), lambda i: (0, 0)),
        ],
        out_specs=[
            pl.BlockSpec((_BLOCK_T, _TOP_K), lambda i: (i, 0)),
            pl.BlockSpec((_BLOCK_T, _TOP_K), lambda i: (i, 0)),
        ],
        out_shape=[
            jax.ShapeDtypeStruct((tokens, _TOP_K), jnp.float32),
            jax.ShapeDtypeStruct((tokens, _TOP_K), jnp.int32),
        ],
        compiler_params=pltpu.CompilerParams(
            dimension_semantics=("arbitrary",),
        ),
    )(hidden_states, weight)
    return vals, idx


# speedup vs baseline: 1.1867x; 1.1867x over previous
import functools

import jax
import jax.numpy as jnp
from jax import lax
from jax.experimental import pallas as pl
from jax.experimental.pallas import tpu as pltpu

_NUM_EXPERTS = 64
_TOP_K = 8
_BLOCK_T = 512


def _router_body(x_ref, w_ref, vals_ref, idx_ref):
    x = x_ref[...]
    w = w_ref[...]
    logits = lax.dot_general(
        x, w, (((1,), (1,)), ((), ())), preferred_element_type=jnp.float32
    )
    u = lax.bitcast_convert_type(logits, jnp.uint32)
    sortable = jnp.where(
        u >= jnp.uint32(0x80000000), ~u, u | jnp.uint32(0x80000000)
    )
    col = lax.broadcasted_iota(jnp.uint32, logits.shape, 1)
    packed = (sortable & jnp.uint32(0xFFFFFFC0)) | (jnp.uint32(63) - col)
    l = lax.bitcast_convert_type(packed ^ jnp.uint32(0x80000000), jnp.int32)
    neg_min = jnp.iinfo(jnp.int32).min
    top_keys = []
    for _ in range(_TOP_K):
        m = jnp.max(l, axis=1, keepdims=True)
        top_keys.append(m)
        l = jnp.where(l == m, neg_min, l)
    top_v = []
    top_i = []
    for m in top_keys:
        ku = lax.bitcast_convert_type(m, jnp.uint32) ^ jnp.uint32(0x80000000)
        top_i.append((jnp.uint32(63) - (ku & jnp.uint32(63))).astype(jnp.int32))
        uu = jnp.where(
            ku >= jnp.uint32(0x80000000), ku ^ jnp.uint32(0x80000000), ~ku
        ) & jnp.uint32(0xFFFFFFC0)
        top_v.append(lax.bitcast_convert_type(uu, jnp.float32))
    exps = [jnp.exp(v - top_v[0]) for v in top_v]
    denom = exps[0]
    for e in exps[1:]:
        denom = denom + e
    inv = 1.0 / denom
    for k in range(_TOP_K):
        vals_ref[:, k : k + 1] = exps[k] * inv
        idx_ref[:, k : k + 1] = top_i[k]


@jax.jit
def kernel(hidden_states, weight):
    tokens, hidden = hidden_states.shape
    grid = (tokens // _BLOCK_T,)
    vals, idx = pl.pallas_call(
        _router_body,
        grid=grid,
        in_specs=[
            pl.BlockSpec((_BLOCK_T, hidden), lambda i: (i, 0)),
            pl.BlockSpec((_NUM_EXPERTS, hidden), lambda i: (0, 0)),
        ],
        out_specs=[
            pl.BlockSpec((_BLOCK_T, _TOP_K), lambda i: (i, 0)),
            pl.BlockSpec((_BLOCK_T, _TOP_K), lambda i: (i, 0)),
        ],
        out_shape=[
            jax.ShapeDtypeStruct((tokens, _TOP_K), jnp.float32),
            jax.ShapeDtypeStruct((tokens, _TOP_K), jnp.int32),
        ],
        compiler_params=pltpu.CompilerParams(
            dimension_semantics=("arbitrary",),
        ),
    )(hidden_states, weight)
    return vals, idx


# speedup vs baseline: 1.7482x; 1.4732x over previous
import functools

import jax
import jax.numpy as jnp
from jax import lax
from jax.experimental import pallas as pl
from jax.experimental.pallas import tpu as pltpu

_NUM_EXPERTS = 64
_TOP_K = 8
_BLOCK_T = 512


def _router_body(x_ref, w_ref, vals_ref, idx_ref):
    x = x_ref[...]
    w = w_ref[...]
    logits = lax.dot_general(
        w, x, (((1,), (1,)), ((), ())), preferred_element_type=jnp.float32
    )
    u = lax.bitcast_convert_type(logits, jnp.uint32)
    sortable = jnp.where(
        u >= jnp.uint32(0x80000000), ~u, u | jnp.uint32(0x80000000)
    )
    row = lax.broadcasted_iota(jnp.uint32, logits.shape, 0)
    packed = (sortable & jnp.uint32(0xFFFFFFC0)) | (jnp.uint32(63) - row)
    l = lax.bitcast_convert_type(packed ^ jnp.uint32(0x80000000), jnp.int32)
    neg_min = jnp.iinfo(jnp.int32).min
    top_keys = []
    for _ in range(_TOP_K):
        m = jnp.max(l, axis=0, keepdims=True)
        top_keys.append(m)
        l = jnp.where(l == m, neg_min, l)
    keys = jnp.concatenate(top_keys, axis=0)
    ku = lax.bitcast_convert_type(keys, jnp.uint32) ^ jnp.uint32(0x80000000)
    top_i = (jnp.uint32(63) - (ku & jnp.uint32(63))).astype(jnp.int32)
    uu = jnp.where(
        ku >= jnp.uint32(0x80000000), ku ^ jnp.uint32(0x80000000), ~ku
    ) & jnp.uint32(0xFFFFFFC0)
    top_v = lax.bitcast_convert_type(uu, jnp.float32)
    e = jnp.exp(top_v - top_v[0:1, :])
    inv = 1.0 / jnp.sum(e, axis=0, keepdims=True)
    vals_ref[...] = jnp.transpose(e * inv)
    idx_ref[...] = jnp.transpose(top_i)


@jax.jit
def kernel(hidden_states, weight):
    tokens, hidden = hidden_states.shape
    grid = (tokens // _BLOCK_T,)
    vals, idx = pl.pallas_call(
        _router_body,
        grid=grid,
        in_specs=[
            pl.BlockSpec((_BLOCK_T, hidden), lambda i: (i, 0)),
            pl.BlockSpec((_NUM_EXPERTS, hidden), lambda i: (0, 0)),
        ],
        out_specs=[
            pl.BlockSpec((_BLOCK_T, _TOP_K), lambda i: (i, 0)),
            pl.BlockSpec((_BLOCK_T, _TOP_K), lambda i: (i, 0)),
        ],
        out_shape=[
            jax.ShapeDtypeStruct((tokens, _TOP_K), jnp.float32),
            jax.ShapeDtypeStruct((tokens, _TOP_K), jnp.int32),
        ],
        compiler_params=pltpu.CompilerParams(
            dimension_semantics=("arbitrary",),
        ),
    )(hidden_states, weight)
    return vals, idx


# speedup vs baseline: 2.4895x; 1.4241x over previous
import functools

import jax
import jax.numpy as jnp
from jax import lax
from jax.experimental import pallas as pl
from jax.experimental.pallas import tpu as pltpu

_NUM_EXPERTS = 64
_TOP_K = 8
_BLOCK_T = 2048


def _router_body(x_ref, w_ref, vals_ref, idx_ref):
    x = x_ref[...]
    w = w_ref[...]
    logits = lax.dot_general(
        w, x, (((1,), (1,)), ((), ())), preferred_element_type=jnp.float32
    )
    u = lax.bitcast_convert_type(logits, jnp.uint32)
    sortable = jnp.where(
        u >= jnp.uint32(0x80000000), ~u, u | jnp.uint32(0x80000000)
    )
    row = lax.broadcasted_iota(jnp.uint32, logits.shape, 0)
    packed = (sortable & jnp.uint32(0xFFFFFFC0)) | (jnp.uint32(63) - row)
    l = lax.bitcast_convert_type(packed ^ jnp.uint32(0x80000000), jnp.int32)
    neg_min = jnp.iinfo(jnp.int32).min
    top_keys = []
    for _ in range(_TOP_K):
        m = jnp.max(l, axis=0, keepdims=True)
        top_keys.append(m)
        l = jnp.where(l == m, neg_min, l)
    keys = jnp.concatenate(top_keys, axis=0)
    ku = lax.bitcast_convert_type(keys, jnp.uint32) ^ jnp.uint32(0x80000000)
    top_i = (jnp.uint32(63) - (ku & jnp.uint32(63))).astype(jnp.int32)
    uu = jnp.where(
        ku >= jnp.uint32(0x80000000), ku ^ jnp.uint32(0x80000000), ~ku
    ) & jnp.uint32(0xFFFFFFC0)
    top_v = lax.bitcast_convert_type(uu, jnp.float32)
    e = jnp.exp(top_v - top_v[0:1, :])
    inv = 1.0 / jnp.sum(e, axis=0, keepdims=True)
    vals_ref[...] = jnp.transpose(e * inv)
    idx_ref[...] = jnp.transpose(top_i)


@jax.jit
def kernel(hidden_states, weight):
    tokens, hidden = hidden_states.shape
    grid = (tokens // _BLOCK_T,)
    vals, idx = pl.pallas_call(
        _router_body,
        grid=grid,
        in_specs=[
            pl.BlockSpec((_BLOCK_T, hidden), lambda i: (i, 0)),
            pl.BlockSpec((_NUM_EXPERTS, hidden), lambda i: (0, 0)),
        ],
        out_specs=[
            pl.BlockSpec((_BLOCK_T, _TOP_K), lambda i: (i, 0)),
            pl.BlockSpec((_BLOCK_T, _TOP_K), lambda i: (i, 0)),
        ],
        out_shape=[
            jax.ShapeDtypeStruct((tokens, _TOP_K), jnp.float32),
            jax.ShapeDtypeStruct((tokens, _TOP_K), jnp.int32),
        ],
        compiler_params=pltpu.CompilerParams(
            dimension_semantics=("arbitrary",),
        ),
    )(hidden_states, weight)
    return vals, idx


# speedup vs baseline: 2.5886x; 1.0398x over previous
import functools

import jax
import jax.numpy as jnp
from jax import lax
from jax.experimental import pallas as pl
from jax.experimental.pallas import tpu as pltpu

_NUM_EXPERTS = 64
_TOP_K = 8
_BLOCK_T = 4096


def _router_body(x_ref, w_ref, vals_ref, idx_ref):
    x = x_ref[...]
    w = w_ref[...]
    logits = lax.dot_general(
        w, x, (((1,), (1,)), ((), ())), preferred_element_type=jnp.float32
    )
    u = lax.bitcast_convert_type(logits, jnp.uint32)
    sortable = jnp.where(
        u >= jnp.uint32(0x80000000), ~u, u | jnp.uint32(0x80000000)
    )
    row = lax.broadcasted_iota(jnp.uint32, logits.shape, 0)
    packed = (sortable & jnp.uint32(0xFFFFFFC0)) | (jnp.uint32(63) - row)
    l = lax.bitcast_convert_type(packed ^ jnp.uint32(0x80000000), jnp.int32)
    neg_min = jnp.iinfo(jnp.int32).min
    top_keys = []
    for _ in range(_TOP_K):
        m = jnp.max(l, axis=0, keepdims=True)
        top_keys.append(m)
        l = jnp.where(l == m, neg_min, l)
    keys = jnp.concatenate(top_keys, axis=0)
    ku = lax.bitcast_convert_type(keys, jnp.uint32) ^ jnp.uint32(0x80000000)
    top_i = (jnp.uint32(63) - (ku & jnp.uint32(63))).astype(jnp.int32)
    uu = jnp.where(
        ku >= jnp.uint32(0x80000000), ku ^ jnp.uint32(0x80000000), ~ku
    ) & jnp.uint32(0xFFFFFFC0)
    top_v = lax.bitcast_convert_type(uu, jnp.float32)
    e = jnp.exp(top_v - top_v[0:1, :])
    inv = 1.0 / jnp.sum(e, axis=0, keepdims=True)
    vals_ref[...] = jnp.transpose(e * inv)
    idx_ref[...] = jnp.transpose(top_i)


@jax.jit
def kernel(hidden_states, weight):
    tokens, hidden = hidden_states.shape
    grid = (tokens // _BLOCK_T,)
    vals, idx = pl.pallas_call(
        _router_body,
        grid=grid,
        in_specs=[
            pl.BlockSpec((_BLOCK_T, hidden), lambda i: (i, 0)),
            pl.BlockSpec((_NUM_EXPERTS, hidden), lambda i: (0, 0)),
        ],
        out_specs=[
            pl.BlockSpec((_BLOCK_T, _TOP_K), lambda i: (i, 0)),
            pl.BlockSpec((_BLOCK_T, _TOP_K), lambda i: (i, 0)),
        ],
        out_shape=[
            jax.ShapeDtypeStruct((tokens, _TOP_K), jnp.float32),
            jax.ShapeDtypeStruct((tokens, _TOP_K), jnp.int32),
        ],
        compiler_params=pltpu.CompilerParams(
            dimension_semantics=("arbitrary",),
        ),
    )(hidden_states, weight)
    return vals, idx


# speedup vs baseline: 2.5936x; 1.0019x over previous
import functools

import jax
import jax.numpy as jnp
from jax import lax
from jax.experimental import pallas as pl
from jax.experimental.pallas import tpu as pltpu

_NUM_EXPERTS = 64
_TOP_K = 8
_BLOCK_T = 4096


def _router_body(x_ref, w_ref, vals_ref, idx_ref):
    x = x_ref[...]
    w = w_ref[...]
    logits = lax.dot_general(
        w, x, (((1,), (1,)), ((), ())), preferred_element_type=jnp.float32
    )
    u = lax.bitcast_convert_type(logits, jnp.uint32)
    sortable = jnp.where(
        u >= jnp.uint32(0x80000000), ~u, u | jnp.uint32(0x80000000)
    )
    row = lax.broadcasted_iota(jnp.uint32, logits.shape, 0)
    packed = (sortable & jnp.uint32(0xFFFFFFC0)) | (jnp.uint32(63) - row)
    l = lax.bitcast_convert_type(packed ^ jnp.uint32(0x80000000), jnp.int32)
    neg_min = jnp.iinfo(jnp.int32).min
    top_keys = []
    for _ in range(_TOP_K):
        m = jnp.max(l, axis=0, keepdims=True)
        top_keys.append(m)
        l = jnp.where(l == m, neg_min, l)
    keys = jnp.concatenate(top_keys, axis=0)
    ku = lax.bitcast_convert_type(keys, jnp.uint32) ^ jnp.uint32(0x80000000)
    top_i = (jnp.uint32(63) - (ku & jnp.uint32(63))).astype(jnp.int32)
    uu = jnp.where(
        ku >= jnp.uint32(0x80000000), ku ^ jnp.uint32(0x80000000), ~ku
    ) & jnp.uint32(0xFFFFFFC0)
    top_v = lax.bitcast_convert_type(uu, jnp.float32)
    e = jnp.exp(top_v - top_v[0:1, :])
    inv = 1.0 / jnp.sum(e, axis=0, keepdims=True)
    vals_ref[...] = jnp.transpose(e * inv)
    idx_ref[...] = jnp.transpose(top_i)


@jax.jit
def kernel(hidden_states, weight):
    tokens, hidden = hidden_states.shape
    grid = (tokens // _BLOCK_T,)
    vals, idx = pl.pallas_call(
        _router_body,
        grid=grid,
        in_specs=[
            pl.BlockSpec((_BLOCK_T, hidden), lambda i: (i, 0)),
            pl.BlockSpec((_NUM_EXPERTS, hidden), lambda i: (0, 0)),
        ],
        out_specs=[
            pl.BlockSpec((_BLOCK_T, _TOP_K), lambda i: (i, 0)),
            pl.BlockSpec((_BLOCK_T, _TOP_K), lambda i: (i, 0)),
        ],
        out_shape=[
            jax.ShapeDtypeStruct((tokens, _TOP_K), jnp.float32),
            jax.ShapeDtypeStruct((tokens, _TOP_K), jnp.int32),
        ],
        compiler_params=pltpu.CompilerParams(
            dimension_semantics=("parallel",),
        ),
    )(hidden_states, weight)
    return vals, idx
